# Initial kernel scaffold; baseline (speedup 1.0000x reference)
import jax
import jax.numpy as jnp
from jax.experimental import pallas as pl


def kernel(seq_len, relative_positions):
    raise NotImplementedError("write your pallas kernel here")


# speedup vs baseline: 13.8923x; 13.8923x over previous
import functools

import jax
import jax.numpy as jnp
from jax import lax
from jax.experimental import pallas as pl
from jax.experimental.pallas import tpu as pltpu
from jax.experimental.pallas import tpu_sc as plsc

_D = 128
_S = 1024
_MAX_LEN = 2048
_T = 2 * _MAX_LEN - 1
_TP = 4096
_FB = 128
_NFB = _TP // _FB

_NC = 2
_NS = 16
_NW = _NC * _NS
_ROWS_PER_W = _S // _NW


def _flip_body(t_ref, o_ref):
    for r in range(_FB):
        o_ref[_FB - 1 - r, :] = t_ref[r, :]


def _make_sc_copy():
    mesh = plsc.VectorSubcoreMesh(core_axis_name="c", subcore_axis_name="s")

    @functools.partial(
        pl.kernel,
        out_type=jax.ShapeDtypeStruct((_S, _S, _D), jnp.float32),
        mesh=mesh,
        scratch_types=[
            pltpu.VMEM_SHARED((_TP, _D), jnp.float32),
        ],
    )
    def sc_copy(flip_hbm, out_hbm, shared):
        c = lax.axis_index("c")
        s = lax.axis_index("s")

        @pl.when(s == 0)
        def _stage():
            pltpu.sync_copy(flip_hbm, shared)

        plsc.subcore_barrier()

        wid = s * _NC + c
        base = wid * _ROWS_PER_W

        def body(r, carry):
            i = base + r
            start = _MAX_LEN - 1 - i
            pltpu.sync_copy(shared.at[pl.ds(start, _S), :], out_hbm.at[i])
            return carry

        lax.fori_loop(0, _ROWS_PER_W, body, 0)

    return sc_copy


_SC_COPY = _make_sc_copy()


def kernel(seq_len, relative_positions):
    del seq_len
    padded = jnp.concatenate(
        [jnp.zeros((1, _D), jnp.float32), relative_positions], axis=0)
    flipped = pl.pallas_call(
        _flip_body,
        grid=(_NFB,),
        in_specs=[pl.BlockSpec((_FB, _D), lambda b: (_NFB - 1 - b, 0))],
        out_specs=pl.BlockSpec((_FB, _D), lambda b: (b, 0)),
        out_shape=jax.ShapeDtypeStruct((_TP, _D), jnp.float32),
    )(padded)
    return _SC_COPY(flipped)


# speedup vs baseline: 13.9729x; 1.0058x over previous
import functools

import jax
import jax.numpy as jnp
from jax import lax
from jax.experimental import pallas as pl
from jax.experimental.pallas import tpu as pltpu
from jax.experimental.pallas import tpu_sc as plsc

_D = 128
_S = 1024
_MAX_LEN = 2048
_T = 2 * _MAX_LEN - 1
_TP = 4096
_FB = 128
_NFB = _TP // _FB

_NC = 2
_NS = 16
_NW = _NC * _NS
_ROWS_PER_W = _S // _NW


def _flip_body(t_ref, o_ref):
    for r in range(_FB):
        o_ref[_FB - 1 - r, :] = t_ref[r, :]


def _make_sc_copy():
    mesh = plsc.VectorSubcoreMesh(core_axis_name="c", subcore_axis_name="s")

    @functools.partial(
        pl.kernel,
        out_type=jax.ShapeDtypeStruct((_S, _S, _D), jnp.float32),
        mesh=mesh,
        scratch_types=[
            pltpu.VMEM_SHARED((_TP, _D), jnp.float32),
            pltpu.SemaphoreType.DMA,
        ],
    )
    def sc_copy(flip_hbm, out_hbm, shared, sem):
        c = lax.axis_index("c")
        s = lax.axis_index("s")

        @pl.when(s == 0)
        def _stage():
            pltpu.sync_copy(flip_hbm, shared)

        plsc.subcore_barrier()

        wid = s * _NC + c
        base = wid * _ROWS_PER_W
        depth = 4

        def body(r, carry):
            i = base + r
            start = _MAX_LEN - 1 - i
            cp = pltpu.make_async_copy(
                shared.at[pl.ds(start, _S), :], out_hbm.at[i], sem)
            cp.start()

            @pl.when(r >= depth - 1)
            def _retire():
                cp.wait()

            return carry

        lax.fori_loop(0, _ROWS_PER_W, body, 0)

        for _ in range(depth - 1):
            pltpu.make_async_copy(
                shared.at[pl.ds(_S, _S), :], out_hbm.at[base], sem).wait()

    return sc_copy


_SC_COPY = _make_sc_copy()


def kernel(seq_len, relative_positions):
    del seq_len
    padded = jnp.concatenate(
        [jnp.zeros((1, _D), jnp.float32), relative_positions], axis=0)
    flipped = pl.pallas_call(
        _flip_body,
        grid=(_NFB,),
        in_specs=[pl.BlockSpec((_FB, _D), lambda b: (_NFB - 1 - b, 0))],
        out_specs=pl.BlockSpec((_FB, _D), lambda b: (b, 0)),
        out_shape=jax.ShapeDtypeStruct((_TP, _D), jnp.float32),
    )(padded)
    return _SC_COPY(flipped)


# speedup vs baseline: 14.8833x; 1.0652x over previous
import functools

import jax
import jax.numpy as jnp
from jax import lax
from jax.experimental import pallas as pl
from jax.experimental.pallas import tpu as pltpu
from jax.experimental.pallas import tpu_sc as plsc

_D = 128
_S = 1024
_MAX_LEN = 2048
_HI = 3070

_NC = 2
_NS = 16
_NW = _NC * _NS
_ROWS_PER_W = _S // _NW

_G = 2048
_CHUNK = _G // _NS
_LANES = 16


def _make_sc_kernel():
    mesh = plsc.VectorSubcoreMesh(core_axis_name="c", subcore_axis_name="s")

    @functools.partial(
        pl.kernel,
        out_type=jax.ShapeDtypeStruct((_S, _S, _D), jnp.float32),
        mesh=mesh,
        scratch_types=[
            pltpu.VMEM_SHARED((_G, _D), jnp.float32),
            pltpu.VMEM((_CHUNK,), jnp.int32),
            pltpu.VMEM((_CHUNK, _D), jnp.float32),
            pltpu.SemaphoreType.DMA,
        ],
    )
    def sc_kernel(table_hbm, out_hbm, shared, idx_v, buf, sem):
        c = lax.axis_index("c")
        s = lax.axis_index("s")

        top = _HI - _CHUNK * s
        for cb in range(_CHUNK // _LANES):
            idx_v[pl.ds(cb * _LANES, _LANES)] = (
                (top - cb * _LANES) - lax.iota(jnp.int32, _LANES))
        pltpu.async_copy(table_hbm.at[idx_v], buf, sem).wait()
        pltpu.sync_copy(buf, shared.at[pl.ds(_CHUNK * s, _CHUNK), :])

        plsc.subcore_barrier()

        wid = s * _NC + c
        base = wid * _ROWS_PER_W
        depth = 4

        def body(r, carry):
            i = base + r
            start = _S - 1 - i
            cp = pltpu.make_async_copy(
                shared.at[pl.ds(start, _S), :], out_hbm.at[i], sem)
            cp.start()

            @pl.when(r >= depth - 1)
            def _retire():
                cp.wait()

            return carry

        lax.fori_loop(0, _ROWS_PER_W, body, 0)

        for _ in range(depth - 1):
            pltpu.make_async_copy(
                shared.at[pl.ds(0, _S), :], out_hbm.at[base], sem).wait()

    return sc_kernel


_SC_KERNEL = _make_sc_kernel()


def kernel(seq_len, relative_positions):
    del seq_len
    return _SC_KERNEL(relative_positions)
